# baseline (speedup 1.0000x reference)
import functools

import jax
import jax.numpy as jnp
from jax import lax
from jax.experimental import pallas as pl
from jax.experimental.pallas import tpu as pltpu
from jax.experimental.pallas import tpu_sc as plsc

N_FIELDS = 3
B = 16384
D = 64
LANES = 16
NC = 2
NS = 16
NW = NC * NS
BPW = B // NW
CHUNK = 128
NCHUNK = BPW // CHUNK


def _newmf_body(it0_hbm, it1_hbm, it2_hbm, table_hbm, out_hbm,
                idx0, idx1, idx2, rows0, rows1, rows2, out_v, sem):
    items_hbm = (it0_hbm, it1_hbm, it2_hbm)
    idx_v = (idx0, idx1, idx2)
    rows_v = (rows0, rows1, rows2)
    wid = lax.axis_index("s") * NC + lax.axis_index("c")
    base = wid * BPW

    for f in range(N_FIELDS):
        pltpu.sync_copy(items_hbm[f].at[pl.ds(base, BPW)], idx_v[f])

    copies = []
    for f in range(N_FIELDS):
        for j in range(NCHUNK):
            copies.append(
                pltpu.async_copy(
                    table_hbm.at[idx_v[f].at[pl.ds(j * CHUNK, CHUNK)]],
                    rows_v[f].at[pl.ds(j * CHUNK, CHUNK)],
                    sem,
                )
            )
    for c in copies:
        c.wait()

    lane = lax.iota(jnp.int32, LANES)
    perms = [jnp.bitwise_xor(lane, 1 << t) for t in range(4)]
    masks = [lane == j for j in range(LANES)]
    dnums = lax.GatherDimensionNumbers(
        offset_dims=(), collapsed_slice_dims=(0,), start_index_map=(0,))

    def _shuffle(v, perm):
        return lax.gather(
            v, perm[:, None], dimension_numbers=dnums, slice_sizes=(1,),
            mode=lax.GatherScatterMode.PROMISE_IN_BOUNDS)

    def grp_body(g, carry):
        row0 = g * LANES
        vec = jnp.zeros((LANES,), jnp.float32)
        for j in range(LANES):
            i = row0 + j
            acc = None
            for k in range(D // LANES):
                sl = pl.ds(k * LANES, LANES)
                p = rows_v[0][i, sl] * rows_v[1][i, sl] * rows_v[2][i, sl]
                acc = p if acc is None else acc + p
            for t in range(4):
                acc = acc + _shuffle(acc, perms[t])
            vec = jnp.where(masks[j], acc, vec)
        out_v[pl.ds(row0, LANES)] = 1.0 / (1.0 + jnp.exp(-vec))
        return carry

    lax.fori_loop(0, BPW // LANES, grp_body, 0)

    pltpu.sync_copy(out_v, out_hbm.at[pl.ds(base, BPW)])


@functools.partial(
    pl.kernel,
    mesh=plsc.VectorSubcoreMesh(core_axis_name="c", subcore_axis_name="s"),
    out_type=jax.ShapeDtypeStruct((B,), jnp.float32),
    compiler_params=pltpu.CompilerParams(use_tc_tiling_on_sc=False),
    scratch_types=[
        pltpu.VMEM((BPW,), jnp.int32),
        pltpu.VMEM((BPW,), jnp.int32),
        pltpu.VMEM((BPW,), jnp.int32),
        pltpu.VMEM((BPW, D), jnp.float32),
        pltpu.VMEM((BPW, D), jnp.float32),
        pltpu.VMEM((BPW, D), jnp.float32),
        pltpu.VMEM((BPW,), jnp.float32),
        pltpu.SemaphoreType.DMA,
    ],
)
def _newmf(it0_hbm, it1_hbm, it2_hbm, table_hbm, out_hbm,
           idx0, idx1, idx2, rows0, rows1, rows2, out_v, sem):
    _newmf_body(it0_hbm, it1_hbm, it2_hbm, table_hbm, out_hbm,
                idx0, idx1, idx2, rows0, rows1, rows2, out_v, sem)


def kernel(items, item_table):
    return _newmf(items[0], items[1], items[2], item_table)
